# baseline (speedup 1.0000x reference)
import functools

import jax
import jax.numpy as jnp
from jax import lax
from jax.experimental import pallas as pl
from jax.experimental.pallas import tpu as pltpu
from jax.experimental.pallas import tpu_sc as plsc

_B = 512
_D_IN = 1024
_D_OUT = 128
_NEG = -1e30
_POS = 1e30

_NC = 2
_NS = 16
_NW = _NC * _NS
_RPW = _B // _NW
_LANES = 16
_CH = _B // _LANES


def _dist_body(x_ref, t_ref, w_ref, b_ref, e_ref):
    h = jnp.dot(x_ref[...], w_ref[...], preferred_element_type=jnp.float32)
    h = h + b_ref[...]
    sq = jnp.sum(h * h, axis=1)
    g = lax.dot_general(
        h, h, (((1,), (1,)), ((), ())), preferred_element_type=jnp.float32
    )
    d2 = jnp.maximum(sq[:, None] + sq[None, :] - 2.0 * g, 0.0)

    t = t_ref[...]
    same = jnp.transpose(t) == t
    ri = lax.broadcasted_iota(jnp.int32, (_B, _B), 0)
    ci = lax.broadcasted_iota(jnp.int32, (_B, _B), 1)
    pos = same & (ri != ci)
    e_ref[...] = jnp.where(
        pos, d2 + 1.0, jnp.where(same, 0.5, -1.0 / (1.0 + d2))
    )


@functools.partial(
    pl.kernel,
    mesh=plsc.VectorSubcoreMesh(core_axis_name="c", subcore_axis_name="s"),
    compiler_params=pltpu.CompilerParams(needs_layout_passes=False),
    out_type=[
        jax.ShapeDtypeStruct((_B,), jnp.float32),
        jax.ShapeDtypeStruct((_B,), jnp.float32),
    ],
    scratch_types=[
        pltpu.VMEM((_RPW, _B), jnp.float32),
        pltpu.VMEM((_RPW * _LANES,), jnp.float32),
        pltpu.VMEM((_RPW * _LANES,), jnp.float32),
        pltpu.VMEM((_RPW,), jnp.float32),
        pltpu.VMEM((_RPW,), jnp.float32),
    ],
)
def _mine(e_hbm, hp_hbm, hn_hbm, e_v, tp_v, tn_v, hp_v, hn_v):
    wid = lax.axis_index("s") * _NC + lax.axis_index("c")
    base = wid * _RPW
    pltpu.sync_copy(e_hbm.at[pl.ds(base, _RPW)], e_v)
    lane16 = lax.iota(jnp.int32, _LANES) * _RPW

    def row_body(r, _):
        v0 = e_v[r, pl.ds(0, _LANES)]
        pacc = v0
        nacc = v0
        for c in range(1, _CH):
            v = e_v[r, pl.ds(c * _LANES, _LANES)]
            pacc = jnp.maximum(pacc, v)
            nacc = jnp.minimum(nacc, v)
        idx = lane16 + r
        plsc.store_scatter(tp_v, [idx], pacc)
        plsc.store_scatter(tn_v, [idx], nacc)
        return 0

    lax.fori_loop(0, _RPW, row_body, 0)
    pmax = tp_v[pl.ds(0, _LANES)]
    nmin = tn_v[pl.ds(0, _LANES)]
    for c in range(1, _LANES):
        pmax = jnp.maximum(pmax, tp_v[pl.ds(c * _LANES, _LANES)])
        nmin = jnp.minimum(nmin, tn_v[pl.ds(c * _LANES, _LANES)])
    hp_v[...] = pmax
    hn_v[...] = nmin
    pltpu.sync_copy(hp_v, hp_hbm.at[pl.ds(base, _RPW)])
    pltpu.sync_copy(hn_v, hn_hbm.at[pl.ds(base, _RPW)])


def _loss_body(hp_ref, hn_ref, out_ref):
    rawp = hp_ref[...]
    rawn = hn_ref[...]
    hp = jnp.where(rawp < 0.99, _NEG, jnp.sqrt(jnp.maximum(rawp - 1.0, 0.0)))
    d2n = -1.0 / jnp.minimum(rawn, -1e-30) - 1.0
    hn = jnp.where(rawn > 0.49, _POS, jnp.sqrt(jnp.maximum(d2n, 0.0)))
    diff = hp - hn
    sp = jnp.log1p(jnp.exp(-jnp.abs(diff))) + jnp.maximum(diff, 0.0)
    out_ref[...] = jnp.sum(sp, axis=1, keepdims=True)


def kernel(inputs, targets, W, b):
    t2 = targets.astype(jnp.int32).reshape(1, _B)
    b2 = b.reshape(1, _D_OUT)
    e = pl.pallas_call(
        _dist_body,
        out_shape=jax.ShapeDtypeStruct((_B, _B), jnp.float32),
    )(inputs, t2, W, b2)
    hp_raw, hn_raw = _mine(e)
    out = pl.pallas_call(
        _loss_body,
        out_shape=jax.ShapeDtypeStruct((1, 1), jnp.float32),
    )(hp_raw.reshape(1, _B), hn_raw.reshape(1, _B))
    return out[0, 0]
